# baseline (speedup 1.0000x reference)
import functools

import jax
import jax.numpy as jnp
from jax import lax
from jax.experimental import pallas as pl
from jax.experimental.pallas import tpu as pltpu
from jax.experimental.pallas import tpu_sc as plsc

INPUT_DIM = 4096
HIDDEN_DIM = 16384
K = 256
SPARSITY_COEF = 0.001

ENC_TILE = 1024
ENC_NT = HIDDEN_DIM // ENC_TILE
SEG = 1024

SPLIT = HIDDEN_DIM
DEC_TILE = 1024
DEC_NT = (HIDDEN_DIM - SPLIT) // DEC_TILE

GCH = 8
NCH = K // GCH
CV = INPUT_DIM // 16
LV = HIDDEN_DIM // 16


def _encode_topk_kernel(x_ref, w_ref, b_ref, lat_ref, pos_ref, enc_scr):
    i = pl.program_id(0)
    acc = jax.lax.dot_general(
        x_ref[...], w_ref[...], (((1,), (1,)), ((), ())),
        preferred_element_type=jnp.float32)
    enc = jnp.clip(acc + b_ref[...], -10.0, 10.0)
    enc_scr[:, pl.ds(i * ENC_TILE, ENC_TILE)] = enc

    @pl.when(i == ENC_NT - 1)
    def _():
        e = enc_scr[...]
        bits = jax.lax.bitcast_convert_type(e, jnp.int32)
        s = jnp.where(bits >= 0, bits, bits ^ jnp.int32(0x7FFFFFFF))
        us = jax.lax.bitcast_convert_type(s, jnp.uint32) ^ jnp.uint32(0x80000000)

        def tbody(b, t):
            cand = t | (jnp.uint32(1) << (31 - b))
            cnt = jnp.sum((us >= cand).astype(jnp.int32), axis=1, keepdims=True)
            return jnp.where(cnt >= K, cand, t)

        t = jax.lax.fori_loop(0, 32, tbody, jnp.zeros((32, 1), jnp.uint32))
        cnt_gt = jnp.sum((us > t).astype(jnp.int32), axis=1, keepdims=True)
        r = K - cnt_gt
        eq = us == t
        iota = jax.lax.broadcasted_iota(jnp.int32, (32, HIDDEN_DIM), 1)

        def jbody(b, J):
            cand = J + (jnp.int32(1) << (14 - b))
            cnt = jnp.sum((eq & (iota < cand)).astype(jnp.int32),
                          axis=1, keepdims=True)
            return jnp.where(cnt < r, cand, J)

        J = jax.lax.fori_loop(0, 15, jbody, jnp.zeros((32, 1), jnp.int32))
        sel = (us > t) | (eq & (iota <= J))
        lat_ref[...] = jnp.where(sel & (e > 0.0), e, 0.0)

        enc_scr[...] = jnp.where(sel & (e > 0.0), 1.0, 0.0)
        ir = jax.lax.broadcasted_iota(jnp.int32, (SEG, SEG), 0)
        ic = jax.lax.broadcasted_iota(jnp.int32, (SEG, SEG), 1)
        ub = jnp.where(ir <= ic, 1.0, 0.0)

        def pbody(g, carry):
            sg = enc_scr[:, pl.ds(g * SEG, SEG)]
            incl = jax.lax.dot_general(
                sg, ub, (((1,), (0,)), ((), ())),
                preferred_element_type=jnp.float32) + carry
            pos_ref[:, pl.ds(g * SEG, SEG)] = incl.astype(jnp.int32) - 1
            return incl[:, SEG - 1:SEG]

        jax.lax.fori_loop(0, HIDDEN_DIM // SEG, pbody,
                          jnp.zeros((32, 1), jnp.float32))


def _sc_decode_kernel(lat_hbm, pos_hbm, w_hbm, bdec_hbm,
                      rec_hbm, abp_hbm,
                      lat_v, pos_v, acc_v, idx_v, val_v,
                      rows0_v, rows1_v, o16_v, sem0, sem1):
    t = lax.axis_index("s") * 2 + lax.axis_index("c")
    pltpu.sync_copy(lat_hbm.at[t], lat_v)
    pltpu.sync_copy(pos_hbm.at[t], pos_v)
    pltpu.sync_copy(bdec_hbm, acc_v)

    zi = jnp.zeros((16,), jnp.int32)
    zf = jnp.zeros((16,), jnp.float32)

    @plsc.parallel_loop(0, (NCH + 1) * GCH // 16, step=1, unroll=4)
    def _(i):
        idx_v[pl.ds(i * 16, 16)] = zi
        val_v[pl.ds(i * 16, 16)] = zf

    iota16 = lax.iota(jnp.int32, 16)
    one16 = iota16 * 0 + 1

    @plsc.parallel_loop(0, LV, step=1, unroll=4, carry=zf)
    def ab_acc(i, ab):
        v = lat_v[pl.ds(i * 16, 16)]
        p = pos_v[pl.ds(i * 16, 16)]
        m = v > 0.0
        plsc.store_scatter(idx_v, [p], iota16 + i * 16, mask=m)
        plsc.store_scatter(val_v, [p], v, mask=m)
        return ab + v

    cnt = jnp.max(pos_v[pl.ds(HIDDEN_DIM - 16, 16)]) + 1
    npair = (cnt + 2 * GCH - 1) // (2 * GCH)

    def gather(ch, rows_ref, sem):
        return pltpu.async_copy(
            w_hbm.at[idx_v.at[pl.ds(ch * GCH, GCH)]], rows_ref, sem)

    def accum(rows_ref, ch):
        vbs = [plsc.load_gather(val_v, [iota16 * 0 + (ch * GCH + j)])
               for j in range(GCH)]

        @plsc.parallel_loop(0, CV, step=1, unroll=4)
        def _(cc):
            sl = pl.ds(cc * 16, 16)
            a = acc_v[sl]
            for j in range(GCH):
                a = a + vbs[j] * rows_ref[j, sl]
            acc_v[sl] = a

    @pl.when(npair > 0)
    def _():
        gather(0, rows0_v, sem0)

    def gbody(p, c):
        c0 = 2 * p
        gather(c0 + 1, rows1_v, sem1)
        pltpu.make_async_copy(
            w_hbm.at[idx_v.at[pl.ds(c0 * GCH, GCH)]], rows0_v, sem0).wait()
        accum(rows0_v, c0)

        @pl.when(p < npair - 1)
        def _():
            gather(c0 + 2, rows0_v, sem0)

        pltpu.make_async_copy(
            w_hbm.at[idx_v.at[pl.ds((c0 + 1) * GCH, GCH)]], rows1_v, sem1).wait()
        accum(rows1_v, c0 + 1)
        return c

    lax.fori_loop(0, npair, gbody, 0)

    pltpu.sync_copy(acc_v, rec_hbm.at[t])
    o16_v[pl.ds(0, 16)] = ab_acc
    pltpu.sync_copy(o16_v, abp_hbm.at[t])


def _sq_kernel(rec_ref, x_ref, sq_ref):
    sq_ref[...] = jnp.sum((rec_ref[...] - x_ref[...]) ** 2).reshape(1, 1)


@functools.partial(jax.jit, static_argnames=())
def kernel(x, W, b_enc, b_dec):
    B, T, C = x.shape
    x_flat = x.reshape(B * T, C)

    latent, pos = pl.pallas_call(
        _encode_topk_kernel,
        grid=(ENC_NT,),
        in_specs=[
            pl.BlockSpec((B * T, C), lambda i: (0, 0)),
            pl.BlockSpec((ENC_TILE, C), lambda i: (i, 0)),
            pl.BlockSpec((1, ENC_TILE), lambda i: (0, i)),
        ],
        out_specs=[
            pl.BlockSpec((B * T, HIDDEN_DIM), lambda i: (0, 0)),
            pl.BlockSpec((B * T, HIDDEN_DIM), lambda i: (0, 0)),
        ],
        out_shape=[
            jax.ShapeDtypeStruct((B * T, HIDDEN_DIM), jnp.float32),
            jax.ShapeDtypeStruct((B * T, HIDDEN_DIM), jnp.int32),
        ],
        scratch_shapes=[pltpu.VMEM((B * T, HIDDEN_DIM), jnp.float32)],
    )(x_flat, W, b_enc.reshape(1, HIDDEN_DIM))

    mesh = plsc.VectorSubcoreMesh(core_axis_name="c", subcore_axis_name="s")
    sc_decode = functools.partial(
        pl.kernel, mesh=mesh,
        compiler_params=pltpu.CompilerParams(needs_layout_passes=False),
        out_type=[
            jax.ShapeDtypeStruct((B * T, C), jnp.float32),
            jax.ShapeDtypeStruct((B * T, 16), jnp.float32),
        ],
        scratch_types=[
            pltpu.VMEM((HIDDEN_DIM,), jnp.float32),
            pltpu.VMEM((HIDDEN_DIM,), jnp.int32),
            pltpu.VMEM((C,), jnp.float32),
            pltpu.VMEM(((NCH + 1) * GCH,), jnp.int32),
            pltpu.VMEM(((NCH + 1) * GCH,), jnp.float32),
            pltpu.VMEM((GCH, C), jnp.float32),
            pltpu.VMEM((GCH, C), jnp.float32),
            pltpu.VMEM((16,), jnp.float32),
            pltpu.SemaphoreType.DMA,
            pltpu.SemaphoreType.DMA,
        ],
    )(_sc_decode_kernel)
    recon, ab_parts = sc_decode(latent, pos, W, b_dec)

    sq_sum = pl.pallas_call(
        _sq_kernel,
        out_shape=jax.ShapeDtypeStruct((1, 1), jnp.float32),
    )(recon, x_flat)

    recon_loss = jnp.minimum(sq_sum[0, 0] / (B * T * C), 100.0)
    sparsity_loss = jnp.minimum(jnp.sum(ab_parts) / (B * T * HIDDEN_DIM), 10.0)
    sae_loss = recon_loss + SPARSITY_COEF * sparsity_loss
    return (recon.reshape(B, T, C), latent.reshape(B, T, HIDDEN_DIM), sae_loss)


# speedup vs baseline: 1.0852x; 1.0852x over previous
import functools

import jax
import jax.numpy as jnp
from jax import lax
from jax.experimental import pallas as pl
from jax.experimental.pallas import tpu as pltpu
from jax.experimental.pallas import tpu_sc as plsc

INPUT_DIM = 4096
HIDDEN_DIM = 16384
K = 256
SPARSITY_COEF = 0.001

ENC_TILE = 1024
ENC_NT = HIDDEN_DIM // ENC_TILE

GCH = 8
NCH = K // GCH
CV = INPUT_DIM // 16
LV = HIDDEN_DIM // 16


def _encode_topk_kernel(x_ref, w_ref, b_ref, lat_ref, enc_scr):
    i = pl.program_id(0)
    acc = jax.lax.dot_general(
        x_ref[...], w_ref[...], (((1,), (1,)), ((), ())),
        preferred_element_type=jnp.float32)
    enc = jnp.clip(acc + b_ref[...], -10.0, 10.0)
    enc_scr[:, pl.ds(i * ENC_TILE, ENC_TILE)] = enc

    @pl.when(i == ENC_NT - 1)
    def _():
        e = enc_scr[...]
        bits = jax.lax.bitcast_convert_type(e, jnp.int32)
        s = jnp.where(bits >= 0, bits, bits ^ jnp.int32(0x7FFFFFFF))
        us = jax.lax.bitcast_convert_type(s, jnp.uint32) ^ jnp.uint32(0x80000000)

        def tbody(b, t):
            cand = t | (jnp.uint32(1) << (31 - b))
            cnt = jnp.sum((us >= cand).astype(jnp.int32), axis=1, keepdims=True)
            return jnp.where(cnt >= K, cand, t)

        t = jax.lax.fori_loop(0, 32, tbody, jnp.zeros((32, 1), jnp.uint32))
        cnt_gt = jnp.sum((us > t).astype(jnp.int32), axis=1, keepdims=True)
        r = K - cnt_gt
        eq = us == t
        iota = jax.lax.broadcasted_iota(jnp.int32, (32, HIDDEN_DIM), 1)

        def jbody(b, J):
            cand = J + (jnp.int32(1) << (14 - b))
            cnt = jnp.sum((eq & (iota < cand)).astype(jnp.int32),
                          axis=1, keepdims=True)
            return jnp.where(cnt < r, cand, J)

        J = jax.lax.fori_loop(0, 15, jbody, jnp.zeros((32, 1), jnp.int32))
        sel = (us > t) | (eq & (iota <= J))
        lat_ref[...] = jnp.where(sel & (e > 0.0), e, 0.0)


def _sc_decode_kernel(lat_hbm, w_hbm, bdec_hbm,
                      rec_hbm, abp_hbm,
                      lat_v, acc_v, idx_v, val_v,
                      rows0_v, rows1_v, rows2_v, o16_v, sem0, sem1, sem2):
    t = lax.axis_index("s") * 2 + lax.axis_index("c")
    pltpu.sync_copy(lat_hbm.at[t], lat_v)
    pltpu.sync_copy(bdec_hbm, acc_v)

    zi = jnp.zeros((16,), jnp.int32)
    zf = jnp.zeros((16,), jnp.float32)

    @plsc.parallel_loop(0, (NCH + 1) * GCH // 16, step=1, unroll=4)
    def _(i):
        idx_v[pl.ds(i * 16, 16)] = zi
        val_v[pl.ds(i * 16, 16)] = zf

    iota16 = lax.iota(jnp.int32, 16)
    one16 = iota16 * 0 + 1

    @plsc.parallel_loop(0, LV, step=1, unroll=4,
                        carry=(jnp.zeros((16,), jnp.int32), zf))
    def _compact(i, carry):
        cntv, ab = carry
        v = lat_v[pl.ds(i * 16, 16)]
        m = v > 0.0
        incl = plsc.cumsum(jnp.where(m, one16, 0))
        pos = cntv + incl - 1
        plsc.store_scatter(idx_v, [pos], iota16 + i * 16, mask=m)
        plsc.store_scatter(val_v, [pos], v, mask=m)
        pop = plsc.all_reduce_population_count(m)
        return cntv + pop, ab + v

    _, ab_acc = _compact

    bufs = (rows0_v, rows1_v, rows2_v)
    sems = (sem0, sem1, sem2)

    def gather(ch, rows_ref, sem):
        return pltpu.async_copy(
            w_hbm.at[idx_v.at[pl.ds(ch * GCH, GCH)]], rows_ref, sem)

    def wait(ch, rows_ref, sem):
        pltpu.make_async_copy(
            w_hbm.at[idx_v.at[pl.ds(ch * GCH, GCH)]], rows_ref, sem).wait()

    def accum(rows_ref, ch):
        vbs = [plsc.load_gather(val_v, [iota16 * 0 + (ch * GCH + j)])
               for j in range(GCH)]

        @plsc.parallel_loop(0, CV, step=1, unroll=4)
        def _(cc):
            sl = pl.ds(cc * 16, 16)
            a = acc_v[sl]
            for j in range(GCH):
                a = a + vbs[j] * rows_ref[j, sl]
            acc_v[sl] = a

    for b in range(3):
        gather(b, bufs[b], sems[b])

    def gbody(p, c):
        for b in range(3):
            ch = 3 * p + b
            wait(ch, bufs[b], sems[b])
            accum(bufs[b], ch)

            @pl.when(ch + 3 < NCH)
            def _():
                gather(ch + 3, bufs[b], sems[b])
        return c

    lax.fori_loop(0, (NCH - 2) // 3, gbody, 0)

    wait(NCH - 2, bufs[0], sems[0])
    accum(bufs[0], NCH - 2)
    wait(NCH - 1, bufs[1], sems[1])
    accum(bufs[1], NCH - 1)

    pltpu.sync_copy(acc_v, rec_hbm.at[t])
    o16_v[pl.ds(0, 16)] = ab_acc
    pltpu.sync_copy(o16_v, abp_hbm.at[t])


def _sq_kernel(rec_ref, x_ref, sq_ref):
    sq_ref[...] = jnp.sum((rec_ref[...] - x_ref[...]) ** 2).reshape(1, 1)


@functools.partial(jax.jit, static_argnames=())
def kernel(x, W, b_enc, b_dec):
    B, T, C = x.shape
    x_flat = x.reshape(B * T, C)

    latent = pl.pallas_call(
        _encode_topk_kernel,
        grid=(ENC_NT,),
        in_specs=[
            pl.BlockSpec((B * T, C), lambda i: (0, 0)),
            pl.BlockSpec((ENC_TILE, C), lambda i: (i, 0)),
            pl.BlockSpec((1, ENC_TILE), lambda i: (0, i)),
        ],
        out_specs=pl.BlockSpec((B * T, HIDDEN_DIM), lambda i: (0, 0)),
        out_shape=jax.ShapeDtypeStruct((B * T, HIDDEN_DIM), jnp.float32),
        scratch_shapes=[pltpu.VMEM((B * T, HIDDEN_DIM), jnp.float32)],
    )(x_flat, W, b_enc.reshape(1, HIDDEN_DIM))

    mesh = plsc.VectorSubcoreMesh(core_axis_name="c", subcore_axis_name="s")
    sc_decode = functools.partial(
        pl.kernel, mesh=mesh,
        compiler_params=pltpu.CompilerParams(needs_layout_passes=False),
        out_type=[
            jax.ShapeDtypeStruct((B * T, C), jnp.float32),
            jax.ShapeDtypeStruct((B * T, 16), jnp.float32),
        ],
        scratch_types=[
            pltpu.VMEM((HIDDEN_DIM,), jnp.float32),
            pltpu.VMEM((C,), jnp.float32),
            pltpu.VMEM(((NCH + 1) * GCH,), jnp.int32),
            pltpu.VMEM(((NCH + 1) * GCH,), jnp.float32),
            pltpu.VMEM((GCH, C), jnp.float32),
            pltpu.VMEM((GCH, C), jnp.float32),
            pltpu.VMEM((GCH, C), jnp.float32),
            pltpu.VMEM((16,), jnp.float32),
            pltpu.SemaphoreType.DMA,
            pltpu.SemaphoreType.DMA,
            pltpu.SemaphoreType.DMA,
        ],
    )(_sc_decode_kernel)
    recon, ab_parts = sc_decode(latent, W, b_dec)

    sq_sum = pl.pallas_call(
        _sq_kernel,
        out_shape=jax.ShapeDtypeStruct((1, 1), jnp.float32),
    )(recon, x_flat)

    recon_loss = jnp.minimum(sq_sum[0, 0] / (B * T * C), 100.0)
    sparsity_loss = jnp.minimum(jnp.sum(ab_parts) / (B * T * HIDDEN_DIM), 10.0)
    sae_loss = recon_loss + SPARSITY_COEF * sparsity_loss
    return (recon.reshape(B, T, C), latent.reshape(B, T, HIDDEN_DIM), sae_loss)


# speedup vs baseline: 1.1028x; 1.0162x over previous
import functools

import jax
import jax.numpy as jnp
from jax import lax
from jax.experimental import pallas as pl
from jax.experimental.pallas import tpu as pltpu
from jax.experimental.pallas import tpu_sc as plsc

INPUT_DIM = 4096
HIDDEN_DIM = 16384
K = 256
SPARSITY_COEF = 0.001

ENC_TILE = 1024
ENC_NT = HIDDEN_DIM // ENC_TILE
SEG = 1024

GCH = 8
NCH = K // GCH
CV = INPUT_DIM // 16
LV = HIDDEN_DIM // 16


def _encode_topk_kernel(x_ref, w_ref, b_ref, lat_ref, enc_scr):
    i = pl.program_id(0)
    acc = jax.lax.dot_general(
        x_ref[...], w_ref[...], (((1,), (1,)), ((), ())),
        preferred_element_type=jnp.float32)
    enc = jnp.clip(acc + b_ref[...], -10.0, 10.0)
    enc_scr[:, pl.ds(i * ENC_TILE, ENC_TILE)] = enc

    @pl.when(i == ENC_NT - 1)
    def _():
        e = enc_scr[...]
        bits = jax.lax.bitcast_convert_type(e, jnp.int32)
        s = jnp.where(bits >= 0, bits, bits ^ jnp.int32(0x7FFFFFFF))
        us = jax.lax.bitcast_convert_type(s, jnp.uint32) ^ jnp.uint32(0x80000000)

        def tbody(b, t):
            cand = t | (jnp.uint32(1) << (31 - b))
            cnt = jnp.sum((us >= cand).astype(jnp.int32), axis=1, keepdims=True)
            return jnp.where(cnt >= K, cand, t)

        t = jax.lax.fori_loop(0, 32, tbody, jnp.zeros((32, 1), jnp.uint32))
        cnt_gt = jnp.sum((us > t).astype(jnp.int32), axis=1, keepdims=True)
        r = K - cnt_gt
        eq = us == t
        sel_gt = us > t

        ir = jax.lax.broadcasted_iota(jnp.int32, (SEG, SEG), 0)
        ic = jax.lax.broadcasted_iota(jnp.int32, (SEG, SEG), 1)
        ub = jnp.where(ir <= ic, 1.0, 0.0)
        r_f = r.astype(jnp.float32)
        carry = jnp.zeros((32, 1), jnp.float32)
        for g in range(HIDDEN_DIM // SEG):
            sl = slice(g * SEG, (g + 1) * SEG)
            eq_g = eq[:, sl]
            incl = jax.lax.dot_general(
                jnp.where(eq_g, 1.0, 0.0), ub, (((1,), (0,)), ((), ())),
                preferred_element_type=jnp.float32) + carry
            sel_g = sel_gt[:, sl] | (eq_g & (incl <= r_f))
            e_g = e[:, sl]
            lat_ref[:, sl] = jnp.where(sel_g & (e_g > 0.0), e_g, 0.0)
            carry = incl[:, SEG - 1:SEG]


def _sc_decode_kernel(lat_hbm, w_hbm, bdec_hbm,
                      rec_hbm, abp_hbm,
                      lat_v, acc_v, idx_v, val_v,
                      rows0_v, rows1_v, rows2_v, o16_v, sem0, sem1, sem2):
    t = lax.axis_index("s") * 2 + lax.axis_index("c")
    pltpu.sync_copy(lat_hbm.at[t], lat_v)
    pltpu.sync_copy(bdec_hbm, acc_v)

    zi = jnp.zeros((16,), jnp.int32)
    zf = jnp.zeros((16,), jnp.float32)

    @plsc.parallel_loop(0, (NCH + 1) * GCH // 16, step=1, unroll=4)
    def _(i):
        idx_v[pl.ds(i * 16, 16)] = zi
        val_v[pl.ds(i * 16, 16)] = zf

    iota16 = lax.iota(jnp.int32, 16)
    one16 = iota16 * 0 + 1

    @plsc.parallel_loop(0, LV, step=1, unroll=4,
                        carry=(jnp.zeros((16,), jnp.int32), zf))
    def _compact(i, carry):
        cntv, ab = carry
        v = lat_v[pl.ds(i * 16, 16)]
        m = v > 0.0
        incl = plsc.cumsum(jnp.where(m, one16, 0))
        pos = cntv + incl - 1
        plsc.store_scatter(idx_v, [pos], iota16 + i * 16, mask=m)
        plsc.store_scatter(val_v, [pos], v, mask=m)
        pop = plsc.all_reduce_population_count(m)
        return cntv + pop, ab + v

    _, ab_acc = _compact

    bufs = (rows0_v, rows1_v, rows2_v)
    sems = (sem0, sem1, sem2)

    def gather(ch, rows_ref, sem):
        return pltpu.async_copy(
            w_hbm.at[idx_v.at[pl.ds(ch * GCH, GCH)]], rows_ref, sem)

    def wait(ch, rows_ref, sem):
        pltpu.make_async_copy(
            w_hbm.at[idx_v.at[pl.ds(ch * GCH, GCH)]], rows_ref, sem).wait()

    def accum(rows_ref, ch):
        vbs = [plsc.load_gather(val_v, [iota16 * 0 + (ch * GCH + j)])
               for j in range(GCH)]

        @plsc.parallel_loop(0, CV, step=1, unroll=4)
        def _(cc):
            sl = pl.ds(cc * 16, 16)
            a = acc_v[sl]
            for j in range(GCH):
                a = a + vbs[j] * rows_ref[j, sl]
            acc_v[sl] = a

    for b in range(3):
        gather(b, bufs[b], sems[b])

    def gbody(p, c):
        for b in range(3):
            ch = 3 * p + b
            wait(ch, bufs[b], sems[b])
            accum(bufs[b], ch)

            @pl.when(ch + 3 < NCH)
            def _():
                gather(ch + 3, bufs[b], sems[b])
        return c

    lax.fori_loop(0, (NCH - 2) // 3, gbody, 0)

    wait(NCH - 2, bufs[0], sems[0])
    accum(bufs[0], NCH - 2)
    wait(NCH - 1, bufs[1], sems[1])
    accum(bufs[1], NCH - 1)

    pltpu.sync_copy(acc_v, rec_hbm.at[t])
    o16_v[pl.ds(0, 16)] = ab_acc
    pltpu.sync_copy(o16_v, abp_hbm.at[t])


def _sq_kernel(rec_ref, x_ref, sq_ref):
    sq_ref[...] = jnp.sum((rec_ref[...] - x_ref[...]) ** 2).reshape(1, 1)


@functools.partial(jax.jit, static_argnames=())
def kernel(x, W, b_enc, b_dec):
    B, T, C = x.shape
    x_flat = x.reshape(B * T, C)

    latent = pl.pallas_call(
        _encode_topk_kernel,
        grid=(ENC_NT,),
        in_specs=[
            pl.BlockSpec((B * T, C), lambda i: (0, 0)),
            pl.BlockSpec((ENC_TILE, C), lambda i: (i, 0)),
            pl.BlockSpec((1, ENC_TILE), lambda i: (0, i)),
        ],
        out_specs=pl.BlockSpec((B * T, HIDDEN_DIM), lambda i: (0, 0)),
        out_shape=jax.ShapeDtypeStruct((B * T, HIDDEN_DIM), jnp.float32),
        scratch_shapes=[pltpu.VMEM((B * T, HIDDEN_DIM), jnp.float32)],
    )(x_flat, W, b_enc.reshape(1, HIDDEN_DIM))

    mesh = plsc.VectorSubcoreMesh(core_axis_name="c", subcore_axis_name="s")
    sc_decode = functools.partial(
        pl.kernel, mesh=mesh,
        compiler_params=pltpu.CompilerParams(needs_layout_passes=False),
        out_type=[
            jax.ShapeDtypeStruct((B * T, C), jnp.float32),
            jax.ShapeDtypeStruct((B * T, 16), jnp.float32),
        ],
        scratch_types=[
            pltpu.VMEM((HIDDEN_DIM,), jnp.float32),
            pltpu.VMEM((C,), jnp.float32),
            pltpu.VMEM(((NCH + 1) * GCH,), jnp.int32),
            pltpu.VMEM(((NCH + 1) * GCH,), jnp.float32),
            pltpu.VMEM((GCH, C), jnp.float32),
            pltpu.VMEM((GCH, C), jnp.float32),
            pltpu.VMEM((GCH, C), jnp.float32),
            pltpu.VMEM((16,), jnp.float32),
            pltpu.SemaphoreType.DMA,
            pltpu.SemaphoreType.DMA,
            pltpu.SemaphoreType.DMA,
        ],
    )(_sc_decode_kernel)
    recon, ab_parts = sc_decode(latent, W, b_dec)

    sq_sum = pl.pallas_call(
        _sq_kernel,
        out_shape=jax.ShapeDtypeStruct((1, 1), jnp.float32),
    )(recon, x_flat)

    recon_loss = jnp.minimum(sq_sum[0, 0] / (B * T * C), 100.0)
    sparsity_loss = jnp.minimum(jnp.sum(ab_parts) / (B * T * HIDDEN_DIM), 10.0)
    sae_loss = recon_loss + SPARSITY_COEF * sparsity_loss
    return (recon.reshape(B, T, C), latent.reshape(B, T, HIDDEN_DIM), sae_loss)
